# baseline (speedup 1.0000x reference)
import jax
import jax.numpy as jnp
from jax.experimental import pallas as pl
from jax.experimental.pallas import tpu as pltpu

_NR, _NC = 2048, 8192
_CR = 64
_NCH = _NR // _CR
_NBUF = 8


def _body(w_hbm, t_ref, o_hbm, ibufs, obufs, isems, osems, s_ref):
    s_ref[:] = jax.nn.sigmoid(t_ref[:])

    def start_in(c):
        k = c % _NBUF
        pltpu.make_async_copy(
            w_hbm.at[pl.ds(c * _CR, _CR), :], ibufs.at[k], isems.at[k]).start()

    for c in range(_NBUF):
        start_in(c)

    for c in range(_NCH):
        k = c % _NBUF
        pltpu.make_async_copy(
            w_hbm.at[pl.ds(c * _CR, _CR), :], ibufs.at[k], isems.at[k]).wait()
        if c >= _NBUF:
            pltpu.make_async_copy(
                obufs.at[k], o_hbm.at[pl.ds((c - _NBUF) * _CR, _CR), :],
                osems.at[k]).wait()
        w = ibufs[k]
        s = s_ref[pl.ds(c * _CR, _CR), :]
        obufs[k] = w - jnp.minimum(jnp.maximum(w, -s), s)
        pltpu.make_async_copy(
            obufs.at[k], o_hbm.at[pl.ds(c * _CR, _CR), :], osems.at[k]).start()
        if c + _NBUF < _NCH:
            start_in(c + _NBUF)

    for c in range(_NCH - _NBUF, _NCH):
        k = c % _NBUF
        pltpu.make_async_copy(
            obufs.at[k], o_hbm.at[pl.ds(c * _CR, _CR), :], osems.at[k]).wait()


def kernel(weight, threshold):
    return pl.pallas_call(
        _body,
        in_specs=[
            pl.BlockSpec(memory_space=pltpu.HBM),
            pl.BlockSpec(memory_space=pltpu.VMEM),
        ],
        out_specs=pl.BlockSpec(memory_space=pltpu.HBM),
        out_shape=jax.ShapeDtypeStruct((_NR, _NC), weight.dtype),
        scratch_shapes=[
            pltpu.VMEM((_NBUF, _CR, _NC), jnp.float32),
            pltpu.VMEM((_NBUF, _CR, _NC), jnp.float32),
            pltpu.SemaphoreType.DMA((_NBUF,)),
            pltpu.SemaphoreType.DMA((_NBUF,)),
            pltpu.VMEM((_NR, 1), jnp.float32),
        ],
    )(weight, threshold)


# speedup vs baseline: 1.0611x; 1.0611x over previous
import jax
import jax.numpy as jnp
from jax.experimental import pallas as pl
from jax.experimental.pallas import tpu as pltpu

_NR, _NC = 2048, 8192
_CR = 64
_NCH = _NR // _CR
_NBUF = 8


def _body(w_hbm, t_ref, o_hbm, ibufs, obufs, isems, osems, s_ref):
    s_ref[:] = jax.nn.sigmoid(t_ref[:]).reshape(_NR, 1)

    def start_in(c):
        k = c % _NBUF
        pltpu.make_async_copy(
            w_hbm.at[pl.ds(c * _CR, _CR), :], ibufs.at[k], isems.at[k]).start()

    for c in range(_NBUF):
        start_in(c)

    for c in range(_NCH):
        k = c % _NBUF
        pltpu.make_async_copy(
            w_hbm.at[pl.ds(c * _CR, _CR), :], ibufs.at[k], isems.at[k]).wait()
        if c >= _NBUF:
            pltpu.make_async_copy(
                obufs.at[k], o_hbm.at[pl.ds((c - _NBUF) * _CR, _CR), :],
                osems.at[k]).wait()
        w = ibufs[k]
        s = s_ref[pl.ds(c * _CR, _CR), :]
        obufs[k] = w - jnp.minimum(jnp.maximum(w, -s), s)
        pltpu.make_async_copy(
            obufs.at[k], o_hbm.at[pl.ds(c * _CR, _CR), :], osems.at[k]).start()
        if c + _NBUF < _NCH:
            start_in(c + _NBUF)

    for c in range(_NCH - _NBUF, _NCH):
        k = c % _NBUF
        pltpu.make_async_copy(
            obufs.at[k], o_hbm.at[pl.ds(c * _CR, _CR), :], osems.at[k]).wait()


def kernel(weight, threshold):
    return pl.pallas_call(
        _body,
        in_specs=[
            pl.BlockSpec(memory_space=pltpu.HBM),
            pl.BlockSpec(memory_space=pltpu.VMEM),
        ],
        out_specs=pl.BlockSpec(memory_space=pltpu.HBM),
        out_shape=jax.ShapeDtypeStruct((_NR, _NC), weight.dtype),
        scratch_shapes=[
            pltpu.VMEM((_NBUF, _CR, _NC), jnp.float32),
            pltpu.VMEM((_NBUF, _CR, _NC), jnp.float32),
            pltpu.SemaphoreType.DMA((_NBUF,)),
            pltpu.SemaphoreType.DMA((_NBUF,)),
            pltpu.VMEM((_NR, 1), jnp.float32),
        ],
    )(weight, threshold.reshape(1, _NR))


# speedup vs baseline: 1.0708x; 1.0092x over previous
import jax
import jax.numpy as jnp
from jax.experimental import pallas as pl
from jax.experimental.pallas import tpu as pltpu

_NR, _NC = 2048, 8192
_CR = 64
_NBUF = 8

_CHUNKS = []
_row = 0
for _r in [16, 16, 16, 16]:
    _CHUNKS.append((_row, _r))
    _row += _r
while _row < _NR - 64:
    _CHUNKS.append((_row, _CR))
    _row += _CR
for _r in [16, 16, 16, 16]:
    _CHUNKS.append((_row, _r))
    _row += _r
assert _row == _NR
_NCH = len(_CHUNKS)


def _body(w_hbm, t_ref, o_hbm, ibufs, obufs, isems, osems, s_ref):
    def in_copy(c):
        row, nr = _CHUNKS[c]
        k = c % _NBUF
        return pltpu.make_async_copy(
            w_hbm.at[pl.ds(row, nr), :], ibufs.at[k, pl.ds(0, nr)],
            isems.at[k])

    def out_copy(c):
        row, nr = _CHUNKS[c]
        k = c % _NBUF
        return pltpu.make_async_copy(
            obufs.at[k, pl.ds(0, nr)], o_hbm.at[pl.ds(row, nr), :],
            osems.at[k])

    for c in range(_NBUF):
        in_copy(c).start()

    s_ref[:] = jax.nn.sigmoid(t_ref[:]).reshape(_NR, 1)

    for c in range(_NCH):
        row, nr = _CHUNKS[c]
        k = c % _NBUF
        in_copy(c).wait()
        if c >= _NBUF:
            out_copy(c - _NBUF).wait()
        w = ibufs[k, pl.ds(0, nr)]
        s = s_ref[pl.ds(row, nr), :]
        obufs[k, pl.ds(0, nr)] = w - jnp.minimum(jnp.maximum(w, -s), s)
        out_copy(c).start()
        if c + _NBUF < _NCH:
            in_copy(c + _NBUF).start()

    for c in range(_NCH - _NBUF, _NCH):
        out_copy(c).wait()


def kernel(weight, threshold):
    return pl.pallas_call(
        _body,
        in_specs=[
            pl.BlockSpec(memory_space=pltpu.HBM),
            pl.BlockSpec(memory_space=pltpu.VMEM),
        ],
        out_specs=pl.BlockSpec(memory_space=pltpu.HBM),
        out_shape=jax.ShapeDtypeStruct((_NR, _NC), weight.dtype),
        scratch_shapes=[
            pltpu.VMEM((_NBUF, _CR, _NC), jnp.float32),
            pltpu.VMEM((_NBUF, _CR, _NC), jnp.float32),
            pltpu.SemaphoreType.DMA((_NBUF,)),
            pltpu.SemaphoreType.DMA((_NBUF,)),
            pltpu.VMEM((_NR, 1), jnp.float32),
        ],
    )(weight, threshold.reshape(1, _NR))


# speedup vs baseline: 1.0709x; 1.0001x over previous
import jax
import jax.numpy as jnp
from jax.experimental import pallas as pl
from jax.experimental.pallas import tpu as pltpu

_NR, _NC = 2048, 8192
_CR = 64
_NBUF = 12

_CHUNKS = []
_row = 0
for _r in [8, 8, 8, 8, 16, 16]:
    _CHUNKS.append((_row, _r))
    _row += _r
while _row < _NR - 64:
    _CHUNKS.append((_row, _CR))
    _row += _CR
for _r in [16, 16, 16, 16]:
    _CHUNKS.append((_row, _r))
    _row += _r
assert _row == _NR
_NCH = len(_CHUNKS)


def _body(w_hbm, t_ref, o_hbm, ibufs, obufs, isems, osems, s_ref):
    def in_copy(c):
        row, nr = _CHUNKS[c]
        k = c % _NBUF
        return pltpu.make_async_copy(
            w_hbm.at[pl.ds(row, nr), :], ibufs.at[k, pl.ds(0, nr)],
            isems.at[k])

    def out_copy(c):
        row, nr = _CHUNKS[c]
        k = c % _NBUF
        return pltpu.make_async_copy(
            obufs.at[k, pl.ds(0, nr)], o_hbm.at[pl.ds(row, nr), :],
            osems.at[k])

    for c in range(_NBUF):
        in_copy(c).start()

    s_ref[:] = jax.nn.sigmoid(t_ref[:]).reshape(_NR, 1)

    for c in range(_NCH):
        row, nr = _CHUNKS[c]
        k = c % _NBUF
        in_copy(c).wait()
        if c >= _NBUF:
            out_copy(c - _NBUF).wait()
        w = ibufs[k, pl.ds(0, nr)]
        s = s_ref[pl.ds(row, nr), :]
        obufs[k, pl.ds(0, nr)] = w - jnp.minimum(jnp.maximum(w, -s), s)
        out_copy(c).start()
        if c + _NBUF < _NCH:
            in_copy(c + _NBUF).start()

    for c in range(_NCH - _NBUF, _NCH):
        out_copy(c).wait()


def kernel(weight, threshold):
    return pl.pallas_call(
        _body,
        in_specs=[
            pl.BlockSpec(memory_space=pltpu.HBM),
            pl.BlockSpec(memory_space=pltpu.VMEM),
        ],
        out_specs=pl.BlockSpec(memory_space=pltpu.HBM),
        out_shape=jax.ShapeDtypeStruct((_NR, _NC), weight.dtype),
        scratch_shapes=[
            pltpu.VMEM((_NBUF, _CR, _NC), jnp.float32),
            pltpu.VMEM((_NBUF, _CR, _NC), jnp.float32),
            pltpu.SemaphoreType.DMA((_NBUF,)),
            pltpu.SemaphoreType.DMA((_NBUF,)),
            pltpu.VMEM((_NR, 1), jnp.float32),
        ],
    )(weight, threshold.reshape(1, _NR))
